# Initial kernel scaffold; baseline (speedup 1.0000x reference)
import jax
import jax.numpy as jnp
from jax.experimental import pallas as pl


def kernel(clean, noisy, t, w1_2d, b1, temb1, w2_big, b2, temb2, alphas):
    raise NotImplementedError("write your pallas kernel here")


# speedup vs baseline: 1.9806x; 1.9806x over previous
import functools

import numpy as np
import jax
import jax.numpy as jnp
from jax.experimental import pallas as pl
from jax.experimental.pallas import tpu as pltpu


def _pair_kernel(alpha_fd_ref, alphas_ref, clean_ref, noisy_ref, mask_ref,
                 w1_ref, b1_ref, temb1_ref, w2_ref, b2_ref, temb2_ref,
                 o_ref, taps1_ref, taps2_ref, *, width, timesteps):
    rows = clean_ref.shape[1]
    hw = clean_ref.shape[2]
    hid2 = w1_ref.shape[0]

    a_fd = alpha_fd_ref[0]
    anchor = a_fd * noisy_ref[0] + (1.0 - a_fd) * clean_ref[0]

    w1 = w1_ref[...]
    w2 = w2_ref[...]
    b1 = b1_ref[...]
    temb1 = temb1_ref[...]
    b2 = b2_ref[...]
    temb2 = temb2_ref[...]
    masks = [mask_ref[k] for k in range(9)]
    offs = [dy * width + dx for dy in (-1, 0, 1) for dx in (-1, 0, 1)]

    def build_taps(h, taps_ref, r):
        for k in range(9):
            off = offs[k]
            if off == 0:
                tap = h
            else:
                tap = pltpu.roll(h, shift=(-off) % hw, axis=1) * masks[k]
            taps_ref[k * r:(k + 1) * r, :] = tap

    def step(i, x):
        a_t = alphas_ref[i, 0]
        a_p = alphas_ref[i, 1]
        st = 1.0 - a_t
        sp = 1.0 - a_p
        build_taps(x, taps1_ref, rows)
        hb = jnp.dot(w1, taps1_ref[...], preferred_element_type=jnp.float32)
        v_t = jnp.broadcast_to(b1 + a_t * temb1, (hid2, hw))
        v_p = jnp.broadcast_to(b1 + a_p * temb1, (hid2, hw))
        d = (sp * jnp.maximum(hb + v_p, 0.0)
             - st * jnp.maximum(hb + v_t, 0.0))
        build_taps(d, taps2_ref, hid2)
        r = jnp.dot(w2, taps2_ref[...], preferred_element_type=jnp.float32)
        cb = (a_t - a_p) * b2 + (a_t * a_t - a_t + a_p - a_p * a_p) * temb2
        return x + r + jnp.broadcast_to(cb, (rows, hw)) + (a_p - a_t) * anchor

    x_final = jax.lax.fori_loop(0, timesteps, step, anchor)
    o_ref[...] = x_final[None].astype(o_ref.dtype)


def _make_tap_masks(h, w):
    ys, xs = np.divmod(np.arange(h * w), w)
    masks = np.zeros((9, 1, h * w), np.float32)
    k = 0
    for dy in (-1, 0, 1):
        for dx in (-1, 0, 1):
            valid = ((ys + dy >= 0) & (ys + dy < h) &
                     (xs + dx >= 0) & (xs + dx < w))
            masks[k, 0, :] = valid
            k += 1
    return jnp.asarray(masks)


def kernel(clean, noisy, t, w1_2d, b1, temb1, w2_big, b2, temb2, alphas):
    n, c, h, w = clean.shape
    hw = h * w
    hid = b1.shape[0]
    timesteps = alphas.shape[0]
    rows = 2 * c
    np2 = n // 2

    f32 = jnp.float32
    clean2 = clean.reshape(np2, rows, hw).astype(f32)
    noisy2 = noisy.reshape(np2, rows, hw).astype(f32)
    alpha_fd = jnp.reshape(jnp.asarray(t, f32) / timesteps, (1,))
    masks = _make_tap_masks(h, w)

    w1r = w1_2d.reshape(hid, 9, rows)[:, :, :c]
    z1 = jnp.zeros_like(w1r)
    w1p = jnp.concatenate([
        jnp.concatenate([w1r, z1], axis=2).reshape(hid, 9 * rows),
        jnp.concatenate([z1, w1r], axis=2).reshape(hid, 9 * rows),
    ], axis=0)

    w2c = w2_big[:c].reshape(c, 9, 2 * hid)[:, :, :hid]
    z2 = jnp.zeros_like(w2c)
    w2p = jnp.concatenate([
        jnp.concatenate([w2c, z2], axis=2).reshape(c, 9 * 2 * hid),
        jnp.concatenate([z2, w2c], axis=2).reshape(c, 9 * 2 * hid),
    ], axis=0)

    b1p = jnp.concatenate([b1, b1], axis=0)
    temb1p = jnp.concatenate([temb1, temb1], axis=0)
    b2p = jnp.concatenate([b2[:c], b2[:c]], axis=0)
    temb2p = jnp.concatenate([temb2[:c], temb2[:c]], axis=0)

    img_spec = pl.BlockSpec((1, rows, hw), lambda i: (i, 0, 0))
    smem = pl.BlockSpec(memory_space=pltpu.MemorySpace.SMEM)

    def full(shape):
        return pl.BlockSpec(shape, lambda i: (0,) * len(shape))

    kern = functools.partial(_pair_kernel, width=w, timesteps=timesteps)
    out = pl.pallas_call(
        kern,
        out_shape=jax.ShapeDtypeStruct((np2, rows, hw), f32),
        grid=(np2,),
        in_specs=[
            smem,
            smem,
            img_spec,
            img_spec,
            full((9, 1, hw)),
            full((2 * hid, 9 * rows)),
            full((2 * hid, 1)),
            full((2 * hid, 1)),
            full((rows, 9 * 2 * hid)),
            full((rows, 1)),
            full((rows, 1)),
        ],
        out_specs=img_spec,
        scratch_shapes=[
            pltpu.VMEM((9 * rows, hw), jnp.float32),
            pltpu.VMEM((9 * 2 * hid, hw), jnp.float32),
        ],
        compiler_params=pltpu.CompilerParams(
            dimension_semantics=("parallel",)),
    )(alpha_fd, alphas, clean2, noisy2, masks,
      w1p, b1p, temb1p, w2p, b2p, temb2p)

    return out.reshape(n, c, h, w)


# speedup vs baseline: 2.3841x; 1.2037x over previous
import functools

import numpy as np
import jax
import jax.numpy as jnp
from jax.experimental import pallas as pl
from jax.experimental.pallas import tpu as pltpu


def _pair_kernel(alpha_fd_ref, alphas_ref, clean_ref, noisy_ref, mask_ref,
                 wg1_ref, b1_ref, temb1_ref, wg2_ref, b2_ref, temb2_ref,
                 o_ref, s1_ref, s2_ref, *, width, timesteps):
    rows = clean_ref.shape[1]
    hw = clean_ref.shape[2]
    hid2 = 2 * (wg1_ref.shape[0] // 6)

    a_fd = alpha_fd_ref[0]
    anchor = a_fd * noisy_ref[0] + (1.0 - a_fd) * clean_ref[0]

    wg1 = wg1_ref[...]
    wg2 = wg2_ref[...]
    b1 = b1_ref[...]
    temb1 = temb1_ref[...]
    b2 = b2_ref[...]
    temb2 = temb2_ref[...]
    mask_x0 = mask_ref[0]
    mask_x1 = mask_ref[1]
    mask_y0 = mask_ref[2]
    mask_y1 = mask_ref[3]

    def conv(h, s_ref, wg, oc):
        r = h.shape[0]
        s_ref[0:r, :] = pltpu.roll(h, shift=1, axis=1) * mask_x0
        s_ref[r:2 * r, :] = h
        s_ref[2 * r:3 * r, :] = pltpu.roll(h, shift=hw - 1, axis=1) * mask_x1
        g = jnp.dot(wg, s_ref[...], preferred_element_type=jnp.float32)
        return (g[oc:2 * oc, :]
                + pltpu.roll(g[0:oc, :], shift=width, axis=1) * mask_y0
                + pltpu.roll(g[2 * oc:3 * oc, :], shift=hw - width, axis=1)
                * mask_y1)

    def step(i, x):
        a_t = alphas_ref[i, 0]
        a_p = alphas_ref[i, 1]
        st = 1.0 - a_t
        sp = 1.0 - a_p
        hb = conv(x, s1_ref, wg1, hid2)
        d = (sp * jnp.maximum(hb + (b1 + a_p * temb1), 0.0)
             - st * jnp.maximum(hb + (b1 + a_t * temb1), 0.0))
        r = conv(d, s2_ref, wg2, rows)
        cb = (a_t - a_p) * b2 + (a_t * a_t - a_t + a_p - a_p * a_p) * temb2
        return x + (r + cb) + (a_p - a_t) * anchor

    x_final = jax.lax.fori_loop(0, timesteps, step, anchor)
    o_ref[...] = x_final[None].astype(o_ref.dtype)


def _make_edge_masks(h, w):
    ys, xs = np.divmod(np.arange(h * w), w)
    masks = np.stack([xs - 1 >= 0, xs + 1 < w, ys - 1 >= 0, ys + 1 < h]
                     ).astype(np.float32)[:, None, :]
    return jnp.asarray(masks)


def kernel(clean, noisy, t, w1_2d, b1, temb1, w2_big, b2, temb2, alphas):
    n, c, h, w = clean.shape
    hw = h * w
    hid = b1.shape[0]
    timesteps = alphas.shape[0]
    rows = 2 * c
    np2 = n // 2

    f32 = jnp.float32
    clean2 = clean.reshape(np2, rows, hw).astype(f32)
    noisy2 = noisy.reshape(np2, rows, hw).astype(f32)
    alpha_fd = jnp.reshape(jnp.asarray(t, f32) / timesteps, (1,))
    masks = _make_edge_masks(h, w)

    w1t = jnp.transpose(w1_2d.reshape(hid, 3, 3, rows)[:, :, :, :c],
                        (1, 0, 2, 3))
    zc = jnp.zeros_like(w1t)
    wg1 = jnp.concatenate([
        jnp.concatenate([w1t, zc], axis=3),
        jnp.concatenate([zc, w1t], axis=3),
    ], axis=1).reshape(3 * 2 * hid, 3 * rows)

    w2t = jnp.transpose(w2_big[:c].reshape(c, 3, 3, 2 * hid)[:, :, :, :hid],
                        (1, 0, 2, 3))
    zc2 = jnp.zeros_like(w2t)
    wg2 = jnp.concatenate([
        jnp.concatenate([w2t, zc2], axis=3),
        jnp.concatenate([zc2, w2t], axis=3),
    ], axis=1).reshape(3 * rows, 3 * 2 * hid)

    b1p = jnp.concatenate([b1, b1], axis=0)
    temb1p = jnp.concatenate([temb1, temb1], axis=0)
    b2p = jnp.concatenate([b2[:c], b2[:c]], axis=0)
    temb2p = jnp.concatenate([temb2[:c], temb2[:c]], axis=0)

    img_spec = pl.BlockSpec((1, rows, hw), lambda i: (i, 0, 0))
    smem = pl.BlockSpec(memory_space=pltpu.MemorySpace.SMEM)

    def full(shape):
        return pl.BlockSpec(shape, lambda i: (0,) * len(shape))

    kern = functools.partial(_pair_kernel, width=w, timesteps=timesteps)
    out = pl.pallas_call(
        kern,
        out_shape=jax.ShapeDtypeStruct((np2, rows, hw), f32),
        grid=(np2,),
        in_specs=[
            smem,
            smem,
            img_spec,
            img_spec,
            full((4, 1, hw)),
            full((6 * hid, 3 * rows)),
            full((2 * hid, 1)),
            full((2 * hid, 1)),
            full((3 * rows, 6 * hid)),
            full((rows, 1)),
            full((rows, 1)),
        ],
        out_specs=img_spec,
        scratch_shapes=[
            pltpu.VMEM((3 * rows, hw), jnp.float32),
            pltpu.VMEM((6 * hid, hw), jnp.float32),
        ],
        compiler_params=pltpu.CompilerParams(
            dimension_semantics=("parallel",)),
    )(alpha_fd, alphas, clean2, noisy2, masks,
      wg1, b1p, temb1p, wg2, b2p, temb2p)

    return out.reshape(n, c, h, w)


# speedup vs baseline: 2.6222x; 1.0999x over previous
import functools

import numpy as np
import jax
import jax.numpy as jnp
from jax.experimental import pallas as pl
from jax.experimental.pallas import tpu as pltpu


def _pair_kernel(alpha_fd_ref, alphas_ref, clean_ref, noisy_ref, mask_ref,
                 wg1_ref, b1_ref, temb1_ref, wg2_ref, b2_ref, temb2_ref,
                 o_ref, s1_ref, s2_ref, hb_ref, r_ref, *, width, timesteps):
    rows = clean_ref.shape[1]
    hw = clean_ref.shape[2]
    hid2 = 2 * (wg1_ref.shape[0] // 6)

    a_fd = alpha_fd_ref[0]
    anchor = a_fd * noisy_ref[0] + (1.0 - a_fd) * clean_ref[0]

    wg1 = wg1_ref[...]
    wg2 = wg2_ref[...]
    b1 = b1_ref[...]
    temb1 = temb1_ref[...]
    b2 = b2_ref[...]
    temb2 = temb2_ref[...]
    mask_x0 = mask_ref[0]
    mask_x1 = mask_ref[1]

    def conv(h, s_ref, wg, oc, out_ref):
        r = h.shape[0]
        s_ref[0:r, :] = pltpu.roll(h, shift=1, axis=1) * mask_x0
        s_ref[r:2 * r, :] = h
        s_ref[2 * r:3 * r, :] = pltpu.roll(h, shift=hw - 1, axis=1) * mask_x1
        g = jnp.dot(wg, s_ref[...], preferred_element_type=jnp.float32)
        out_ref[...] = (g[oc:2 * oc, :]
                        + pltpu.roll(g[0:oc, :], shift=width, axis=1)
                        + pltpu.roll(g[2 * oc:3 * oc, :],
                                     shift=hw - width, axis=1))
        out_ref[:, 0:width] = (g[oc:2 * oc, 0:width]
                               + g[2 * oc:3 * oc, width:2 * width])
        out_ref[:, hw - width:hw] = (g[oc:2 * oc, hw - width:hw]
                                     + g[0:oc, hw - 2 * width:hw - width])

    def step(i, x):
        a_t = alphas_ref[i, 0]
        a_p = alphas_ref[i, 1]
        st = 1.0 - a_t
        sp = 1.0 - a_p
        conv(x, s1_ref, wg1, hid2, hb_ref)
        hb = hb_ref[...]
        d = (sp * jnp.maximum(hb + (b1 + a_p * temb1), 0.0)
             - st * jnp.maximum(hb + (b1 + a_t * temb1), 0.0))
        conv(d, s2_ref, wg2, rows, r_ref)
        cb = (a_t - a_p) * b2 + (a_t * a_t - a_t + a_p - a_p * a_p) * temb2
        return x + (r_ref[...] + cb) + (a_p - a_t) * anchor

    x_final = jax.lax.fori_loop(0, timesteps, step, anchor)
    o_ref[...] = x_final[None].astype(o_ref.dtype)


def _make_edge_masks(h, w):
    ys, xs = np.divmod(np.arange(h * w), w)
    masks = np.stack([xs - 1 >= 0, xs + 1 < w, ys - 1 >= 0, ys + 1 < h]
                     ).astype(np.float32)[:, None, :]
    return jnp.asarray(masks)


def kernel(clean, noisy, t, w1_2d, b1, temb1, w2_big, b2, temb2, alphas):
    n, c, h, w = clean.shape
    hw = h * w
    hid = b1.shape[0]
    timesteps = alphas.shape[0]
    rows = 2 * c
    np2 = n // 2

    f32 = jnp.float32
    clean2 = clean.reshape(np2, rows, hw).astype(f32)
    noisy2 = noisy.reshape(np2, rows, hw).astype(f32)
    alpha_fd = jnp.reshape(jnp.asarray(t, f32) / timesteps, (1,))
    masks = _make_edge_masks(h, w)

    w1t = jnp.transpose(w1_2d.reshape(hid, 3, 3, rows)[:, :, :, :c],
                        (1, 0, 2, 3))
    zc = jnp.zeros_like(w1t)
    wg1 = jnp.concatenate([
        jnp.concatenate([w1t, zc], axis=3),
        jnp.concatenate([zc, w1t], axis=3),
    ], axis=1).reshape(3 * 2 * hid, 3 * rows)

    w2t = jnp.transpose(w2_big[:c].reshape(c, 3, 3, 2 * hid)[:, :, :, :hid],
                        (1, 0, 2, 3))
    zc2 = jnp.zeros_like(w2t)
    wg2 = jnp.concatenate([
        jnp.concatenate([w2t, zc2], axis=3),
        jnp.concatenate([zc2, w2t], axis=3),
    ], axis=1).reshape(3 * rows, 3 * 2 * hid)

    b1p = jnp.concatenate([b1, b1], axis=0)
    temb1p = jnp.concatenate([temb1, temb1], axis=0)
    b2p = jnp.concatenate([b2[:c], b2[:c]], axis=0)
    temb2p = jnp.concatenate([temb2[:c], temb2[:c]], axis=0)

    img_spec = pl.BlockSpec((1, rows, hw), lambda i: (i, 0, 0))
    smem = pl.BlockSpec(memory_space=pltpu.MemorySpace.SMEM)

    def full(shape):
        return pl.BlockSpec(shape, lambda i: (0,) * len(shape))

    kern = functools.partial(_pair_kernel, width=w, timesteps=timesteps)
    out = pl.pallas_call(
        kern,
        out_shape=jax.ShapeDtypeStruct((np2, rows, hw), f32),
        grid=(np2,),
        in_specs=[
            smem,
            smem,
            img_spec,
            img_spec,
            full((4, 1, hw)),
            full((6 * hid, 3 * rows)),
            full((2 * hid, 1)),
            full((2 * hid, 1)),
            full((3 * rows, 6 * hid)),
            full((rows, 1)),
            full((rows, 1)),
        ],
        out_specs=img_spec,
        scratch_shapes=[
            pltpu.VMEM((3 * rows, hw), jnp.float32),
            pltpu.VMEM((6 * hid, hw), jnp.float32),
            pltpu.VMEM((2 * hid, hw), jnp.float32),
            pltpu.VMEM((rows, hw), jnp.float32),
        ],
        compiler_params=pltpu.CompilerParams(
            dimension_semantics=("parallel",)),
    )(alpha_fd, alphas, clean2, noisy2, masks,
      wg1, b1p, temb1p, wg2, b2p, temb2p)

    return out.reshape(n, c, h, w)


# speedup vs baseline: 2.6602x; 1.0145x over previous
import functools

import numpy as np
import jax
import jax.numpy as jnp
from jax.experimental import pallas as pl
from jax.experimental.pallas import tpu as pltpu


def _pair_kernel(alpha_fd_ref, alphas_ref, clean_ref, noisy_ref, mask_ref,
                 wg1_ref, b1_ref, temb1_ref, wg2_ref, b2_ref, temb2_ref,
                 o_ref, s1_ref, s2_ref, hb_ref, r_ref, *, width, timesteps):
    rows = clean_ref.shape[1]
    hw = clean_ref.shape[2]
    hid2 = wg1_ref.shape[0] // 3

    a_fd = alpha_fd_ref[0]
    anchor = a_fd * noisy_ref[0] + (1.0 - a_fd) * clean_ref[0]

    wg1 = wg1_ref[...]
    wg2 = wg2_ref[...]
    b1 = b1_ref[...]
    temb1 = temb1_ref[...]
    b2 = b2_ref[...]
    temb2 = temb2_ref[...]
    mask_x0 = mask_ref[0]
    mask_x1 = mask_ref[1]

    def conv(h, s_ref, wg, oc, out_ref):
        r = h.shape[0]
        s_ref[0:r, :] = pltpu.roll(h, shift=1, axis=1) * mask_x0
        s_ref[r:2 * r, :] = h
        s_ref[2 * r:3 * r, :] = pltpu.roll(h, shift=hw - 1, axis=1) * mask_x1
        g = jnp.dot(wg, s_ref[...], preferred_element_type=jnp.float32)
        out_ref[...] = (g[oc:2 * oc, :]
                        + pltpu.roll(g[0:oc, :], shift=width, axis=1)
                        + pltpu.roll(g[2 * oc:3 * oc, :],
                                     shift=hw - width, axis=1))
        out_ref[:, 0:width] = (g[oc:2 * oc, 0:width]
                               + g[2 * oc:3 * oc, width:2 * width])
        out_ref[:, hw - width:hw] = (g[oc:2 * oc, hw - width:hw]
                                     + g[0:oc, hw - 2 * width:hw - width])

    def step(i, x):
        a_t = alphas_ref[i, 0]
        a_p = alphas_ref[i, 1]
        st = 1.0 - a_t
        sp = 1.0 - a_p
        conv(x, s1_ref, wg1, hid2, hb_ref)
        hb = hb_ref[...]
        d = (sp * jnp.maximum(hb + (b1 + a_p * temb1), 0.0)
             - st * jnp.maximum(hb + (b1 + a_t * temb1), 0.0))
        conv(d, s2_ref, wg2, rows, r_ref)
        cb = (a_t - a_p) * b2 + (a_t * a_t - a_t + a_p - a_p * a_p) * temb2
        return x + (r_ref[...] + cb) + (a_p - a_t) * anchor

    x_final = jax.lax.fori_loop(0, timesteps, step, anchor)
    o_ref[...] = x_final[None].astype(o_ref.dtype)


def _make_edge_masks(h, w):
    ys, xs = np.divmod(np.arange(h * w), w)
    masks = np.stack([xs - 1 >= 0, xs + 1 < w, ys - 1 >= 0, ys + 1 < h]
                     ).astype(np.float32)[:, None, :]
    return jnp.asarray(masks)


def kernel(clean, noisy, t, w1_2d, b1, temb1, w2_big, b2, temb2, alphas):
    n, c, h, w = clean.shape
    hw = h * w
    hid = b1.shape[0]
    timesteps = alphas.shape[0]
    P = 4
    rows = P * c
    np2 = n // P

    f32 = jnp.float32
    clean2 = clean.reshape(np2, rows, hw).astype(f32)
    noisy2 = noisy.reshape(np2, rows, hw).astype(f32)
    alpha_fd = jnp.reshape(jnp.asarray(t, f32) / timesteps, (1,))
    masks = _make_edge_masks(h, w)

    w1t = jnp.transpose(w1_2d.reshape(hid, 3, 3, 2 * c)[:, :, :, :c],
                        (1, 0, 2, 3))
    wg1_6d = jnp.zeros((3, P, hid, 3, P, c), f32)
    for j in range(P):
        wg1_6d = wg1_6d.at[:, j, :, :, j, :].set(w1t)
    wg1 = wg1_6d.reshape(3 * P * hid, 3 * P * c)

    w2t = jnp.transpose(w2_big[:c].reshape(c, 3, 3, 2 * hid)[:, :, :, :hid],
                        (1, 0, 2, 3))
    wg2_6d = jnp.zeros((3, P, c, 3, P, hid), f32)
    for j in range(P):
        wg2_6d = wg2_6d.at[:, j, :, :, j, :].set(w2t)
    wg2 = wg2_6d.reshape(3 * P * c, 3 * P * hid)

    b1p = jnp.tile(b1, (P, 1))
    temb1p = jnp.tile(temb1, (P, 1))
    b2p = jnp.tile(b2[:c], (P, 1))
    temb2p = jnp.tile(temb2[:c], (P, 1))

    img_spec = pl.BlockSpec((1, rows, hw), lambda i: (i, 0, 0))
    smem = pl.BlockSpec(memory_space=pltpu.MemorySpace.SMEM)

    def full(shape):
        return pl.BlockSpec(shape, lambda i: (0,) * len(shape))

    kern = functools.partial(_pair_kernel, width=w, timesteps=timesteps)
    out = pl.pallas_call(
        kern,
        out_shape=jax.ShapeDtypeStruct((np2, rows, hw), f32),
        grid=(np2,),
        in_specs=[
            smem,
            smem,
            img_spec,
            img_spec,
            full((4, 1, hw)),
            full((3 * P * hid, 3 * rows)),
            full((P * hid, 1)),
            full((P * hid, 1)),
            full((3 * rows, 3 * P * hid)),
            full((rows, 1)),
            full((rows, 1)),
        ],
        out_specs=img_spec,
        scratch_shapes=[
            pltpu.VMEM((3 * rows, hw), jnp.float32),
            pltpu.VMEM((3 * P * hid, hw), jnp.float32),
            pltpu.VMEM((P * hid, hw), jnp.float32),
            pltpu.VMEM((rows, hw), jnp.float32),
        ],
        compiler_params=pltpu.CompilerParams(
            dimension_semantics=("parallel",)),
    )(alpha_fd, alphas, clean2, noisy2, masks,
      wg1, b1p, temb1p, wg2, b2p, temb2p)

    return out.reshape(n, c, h, w)


# speedup vs baseline: 2.6984x; 1.0144x over previous
import functools

import numpy as np
import jax
import jax.numpy as jnp
from jax.experimental import pallas as pl
from jax.experimental.pallas import tpu as pltpu


def _pair_kernel(alpha_fd_ref, alphas_ref, clean_ref, noisy_ref, mask_ref,
                 wg1_ref, b1_ref, temb1_ref, wg2_ref, b2_ref, temb2_ref,
                 o_ref, s1_ref, s2_ref, hb_ref, x_ref, *, width, timesteps):
    rows = clean_ref.shape[1]
    hw = clean_ref.shape[2]
    hid2 = wg1_ref.shape[0] // 3

    a_fd = alpha_fd_ref[0]
    anchor = a_fd * noisy_ref[0] + (1.0 - a_fd) * clean_ref[0]

    wg1 = wg1_ref[...]
    wg2 = wg2_ref[...]
    b1 = b1_ref[...]
    temb1 = temb1_ref[...]
    b2 = b2_ref[...]
    temb2 = temb2_ref[...]
    mask_x0 = mask_ref[0]
    mask_x1 = mask_ref[1]

    def conv(h, s_ref, wg, oc, out_ref):
        r = h.shape[0]
        s_ref[0:r, :] = pltpu.roll(h, shift=1, axis=1) * mask_x0
        s_ref[r:2 * r, :] = h
        s_ref[2 * r:3 * r, :] = pltpu.roll(h, shift=hw - 1, axis=1) * mask_x1
        g = jnp.dot(wg, s_ref[...], preferred_element_type=jnp.float32)
        out_ref[...] = (g[oc:2 * oc, :]
                        + pltpu.roll(g[0:oc, :], shift=width, axis=1)
                        + pltpu.roll(g[2 * oc:3 * oc, :],
                                     shift=hw - width, axis=1))
        out_ref[:, 0:width] = (g[oc:2 * oc, 0:width]
                               + g[2 * oc:3 * oc, width:2 * width])
        out_ref[:, hw - width:hw] = (g[oc:2 * oc, hw - width:hw]
                                     + g[0:oc, hw - 2 * width:hw - width])

    def step(i, _):
        a_t = alphas_ref[i, 0]
        a_p = alphas_ref[i, 1]
        st = 1.0 - a_t
        sp = 1.0 - a_p
        x = x_ref[...]
        conv(x, s1_ref, wg1, hid2, hb_ref)
        hb = hb_ref[...]
        d = (sp * jnp.maximum(hb + (b1 + a_p * temb1), 0.0)
             - st * jnp.maximum(hb + (b1 + a_t * temb1), 0.0))
        r = d.shape[0]
        s2_ref[0:r, :] = pltpu.roll(d, shift=1, axis=1) * mask_x0
        s2_ref[r:2 * r, :] = d
        s2_ref[2 * r:3 * r, :] = pltpu.roll(d, shift=hw - 1, axis=1) * mask_x1
        g = jnp.dot(wg2, s2_ref[...], preferred_element_type=jnp.float32)
        cb = (a_t - a_p) * b2 + (a_t * a_t - a_t + a_p - a_p * a_p) * temb2
        base = x + ((g[rows:2 * rows, :] + cb) + (a_p - a_t) * anchor)
        x_ref[...] = (base
                      + pltpu.roll(g[0:rows, :], shift=width, axis=1)
                      + pltpu.roll(g[2 * rows:3 * rows, :],
                                   shift=hw - width, axis=1))
        x_ref[:, 0:width] = (base[:, 0:width]
                             + g[2 * rows:3 * rows, width:2 * width])
        x_ref[:, hw - width:hw] = (base[:, hw - width:hw]
                                   + g[0:rows, hw - 2 * width:hw - width])
        return 0

    x_ref[...] = anchor
    jax.lax.fori_loop(0, timesteps, step, 0)
    o_ref[...] = x_ref[...][None].astype(o_ref.dtype)


def _make_edge_masks(h, w):
    ys, xs = np.divmod(np.arange(h * w), w)
    masks = np.stack([xs - 1 >= 0, xs + 1 < w, ys - 1 >= 0, ys + 1 < h]
                     ).astype(np.float32)[:, None, :]
    return jnp.asarray(masks)


def kernel(clean, noisy, t, w1_2d, b1, temb1, w2_big, b2, temb2, alphas):
    n, c, h, w = clean.shape
    hw = h * w
    hid = b1.shape[0]
    timesteps = alphas.shape[0]
    P = 4
    rows = P * c
    np2 = n // P

    f32 = jnp.float32
    clean2 = clean.reshape(np2, rows, hw).astype(f32)
    noisy2 = noisy.reshape(np2, rows, hw).astype(f32)
    alpha_fd = jnp.reshape(jnp.asarray(t, f32) / timesteps, (1,))
    masks = _make_edge_masks(h, w)

    w1t = jnp.transpose(w1_2d.reshape(hid, 3, 3, 2 * c)[:, :, :, :c],
                        (1, 0, 2, 3))
    wg1_6d = jnp.zeros((3, P, hid, 3, P, c), f32)
    for j in range(P):
        wg1_6d = wg1_6d.at[:, j, :, :, j, :].set(w1t)
    wg1 = wg1_6d.reshape(3 * P * hid, 3 * P * c)

    w2t = jnp.transpose(w2_big[:c].reshape(c, 3, 3, 2 * hid)[:, :, :, :hid],
                        (1, 0, 2, 3))
    wg2_6d = jnp.zeros((3, P, c, 3, P, hid), f32)
    for j in range(P):
        wg2_6d = wg2_6d.at[:, j, :, :, j, :].set(w2t)
    wg2 = wg2_6d.reshape(3 * P * c, 3 * P * hid)

    b1p = jnp.tile(b1, (P, 1))
    temb1p = jnp.tile(temb1, (P, 1))
    b2p = jnp.tile(b2[:c], (P, 1))
    temb2p = jnp.tile(temb2[:c], (P, 1))

    img_spec = pl.BlockSpec((1, rows, hw), lambda i: (i, 0, 0))
    smem = pl.BlockSpec(memory_space=pltpu.MemorySpace.SMEM)

    def full(shape):
        return pl.BlockSpec(shape, lambda i: (0,) * len(shape))

    kern = functools.partial(_pair_kernel, width=w, timesteps=timesteps)
    out = pl.pallas_call(
        kern,
        out_shape=jax.ShapeDtypeStruct((np2, rows, hw), f32),
        grid=(np2,),
        in_specs=[
            smem,
            smem,
            img_spec,
            img_spec,
            full((4, 1, hw)),
            full((3 * P * hid, 3 * rows)),
            full((P * hid, 1)),
            full((P * hid, 1)),
            full((3 * rows, 3 * P * hid)),
            full((rows, 1)),
            full((rows, 1)),
        ],
        out_specs=img_spec,
        scratch_shapes=[
            pltpu.VMEM((3 * rows, hw), jnp.float32),
            pltpu.VMEM((3 * P * hid, hw), jnp.float32),
            pltpu.VMEM((P * hid, hw), jnp.float32),
            pltpu.VMEM((rows, hw), jnp.float32),
        ],
        compiler_params=pltpu.CompilerParams(
            dimension_semantics=("parallel",)),
    )(alpha_fd, alphas, clean2, noisy2, masks,
      wg1, b1p, temb1p, wg2, b2p, temb2p)

    return out.reshape(n, c, h, w)


# speedup vs baseline: 2.9562x; 1.0955x over previous
import functools

import numpy as np
import jax
import jax.numpy as jnp
from jax.experimental import pallas as pl
from jax.experimental.pallas import tpu as pltpu


def _diff_kernel(alpha_fd_ref, alphas_ref, clean_ref, noisy_ref, mask_ref,
                 wg1_ref, b1_ref, temb1_ref, wg2_ref, b2_ref, temb2_ref,
                 o_ref, s1a, s1b, s2a, s2b, hba, hbb, xa, xb,
                 *, width, timesteps):
    gr = xa.shape[0]
    hw = clean_ref.shape[2]
    hid2 = wg1_ref.shape[0] // 3

    a_fd = alpha_fd_ref[0]

    wg1 = wg1_ref[...]
    wg2 = wg2_ref[...]
    b1 = b1_ref[...]
    temb1 = temb1_ref[...]
    b2 = b2_ref[...]
    temb2 = temb2_ref[...]
    mask_x0 = mask_ref[0]
    mask_x1 = mask_ref[1]

    anc_a = (a_fd * noisy_ref[0, 0:gr, :]
             + (1.0 - a_fd) * clean_ref[0, 0:gr, :])
    anc_b = (a_fd * noisy_ref[0, gr:2 * gr, :]
             + (1.0 - a_fd) * clean_ref[0, gr:2 * gr, :])
    groups = ((s1a, s2a, hba, xa, anc_a, 0), (s1b, s2b, hbb, xb, anc_b, gr))

    def conv1(h, s_ref, out_ref):
        s_ref[0:gr, :] = pltpu.roll(h, shift=1, axis=1) * mask_x0
        s_ref[gr:2 * gr, :] = h
        s_ref[2 * gr:3 * gr, :] = pltpu.roll(h, shift=hw - 1, axis=1) * mask_x1
        g = jnp.dot(wg1, s_ref[...], preferred_element_type=jnp.float32)
        out_ref[...] = (g[hid2:2 * hid2, :]
                        + pltpu.roll(g[0:hid2, :], shift=width, axis=1)
                        + pltpu.roll(g[2 * hid2:3 * hid2, :],
                                     shift=hw - width, axis=1))
        out_ref[:, 0:width] = (g[hid2:2 * hid2, 0:width]
                               + g[2 * hid2:3 * hid2, width:2 * width])
        out_ref[:, hw - width:hw] = (g[hid2:2 * hid2, hw - width:hw]
                                     + g[0:hid2, hw - 2 * width:hw - width])

    def step(i, _):
        a_t = alphas_ref[i, 0]
        a_p = alphas_ref[i, 1]
        st = 1.0 - a_t
        sp = 1.0 - a_p
        v_p = b1 + a_p * temb1
        v_t = b1 + a_t * temb1
        cb = (a_t - a_p) * b2 + (a_t * a_t - a_t + a_p - a_p * a_p) * temb2
        for s1_ref, s2_ref, hb_ref, x_ref, anc, ro in groups:
            x = x_ref[...]
            conv1(x, s1_ref, hb_ref)
            hb = hb_ref[...]
            d = (sp * jnp.maximum(hb + v_p, 0.0)
                 - st * jnp.maximum(hb + v_t, 0.0))
            s2_ref[0:hid2, :] = pltpu.roll(d, shift=1, axis=1) * mask_x0
            s2_ref[hid2:2 * hid2, :] = d
            s2_ref[2 * hid2:3 * hid2, :] = (
                pltpu.roll(d, shift=hw - 1, axis=1) * mask_x1)
            g = jnp.dot(wg2, s2_ref[...], preferred_element_type=jnp.float32)
            base = x + ((g[gr:2 * gr, :] + cb) + (a_p - a_t) * anc)
            x_ref[...] = (base
                          + pltpu.roll(g[0:gr, :], shift=width, axis=1)
                          + pltpu.roll(g[2 * gr:3 * gr, :],
                                       shift=hw - width, axis=1))
            x_ref[:, 0:width] = (base[:, 0:width]
                                 + g[2 * gr:3 * gr, width:2 * width])
            x_ref[:, hw - width:hw] = (base[:, hw - width:hw]
                                       + g[0:gr, hw - 2 * width:hw - width])
        return 0

    for _, _, _, x_ref, anc, ro in groups:
        x_ref[...] = anc
    jax.lax.fori_loop(0, timesteps, step, 0)
    for _, _, _, x_ref, anc, ro in groups:
        o_ref[0, ro:ro + gr, :] = x_ref[...].astype(o_ref.dtype)


def _make_edge_masks(h, w):
    ys, xs = np.divmod(np.arange(h * w), w)
    masks = np.stack([xs - 1 >= 0, xs + 1 < w, ys - 1 >= 0, ys + 1 < h]
                     ).astype(np.float32)[:, None, :]
    return jnp.asarray(masks)


def kernel(clean, noisy, t, w1_2d, b1, temb1, w2_big, b2, temb2, alphas):
    n, c, h, w = clean.shape
    hw = h * w
    hid = b1.shape[0]
    timesteps = alphas.shape[0]
    P = 4
    G = 2
    gr = P * c
    rows = G * gr
    nblk = n // (P * G)

    f32 = jnp.float32
    clean2 = clean.reshape(nblk, rows, hw).astype(f32)
    noisy2 = noisy.reshape(nblk, rows, hw).astype(f32)
    alpha_fd = jnp.reshape(jnp.asarray(t, f32) / timesteps, (1,))
    masks = _make_edge_masks(h, w)

    w1t = jnp.transpose(w1_2d.reshape(hid, 3, 3, 2 * c)[:, :, :, :c],
                        (1, 0, 2, 3))
    wg1_6d = jnp.zeros((3, P, hid, 3, P, c), f32)
    for j in range(P):
        wg1_6d = wg1_6d.at[:, j, :, :, j, :].set(w1t)
    wg1 = wg1_6d.reshape(3 * P * hid, 3 * P * c)

    w2t = jnp.transpose(w2_big[:c].reshape(c, 3, 3, 2 * hid)[:, :, :, :hid],
                        (1, 0, 2, 3))
    wg2_6d = jnp.zeros((3, P, c, 3, P, hid), f32)
    for j in range(P):
        wg2_6d = wg2_6d.at[:, j, :, :, j, :].set(w2t)
    wg2 = wg2_6d.reshape(3 * P * c, 3 * P * hid)

    b1p = jnp.tile(b1, (P, 1))
    temb1p = jnp.tile(temb1, (P, 1))
    b2p = jnp.tile(b2[:c], (P, 1))
    temb2p = jnp.tile(temb2[:c], (P, 1))

    img_spec = pl.BlockSpec((1, rows, hw), lambda i: (i, 0, 0))
    smem = pl.BlockSpec(memory_space=pltpu.MemorySpace.SMEM)

    def full(shape):
        return pl.BlockSpec(shape, lambda i: (0,) * len(shape))

    kern = functools.partial(_diff_kernel, width=w, timesteps=timesteps)
    out = pl.pallas_call(
        kern,
        out_shape=jax.ShapeDtypeStruct((nblk, rows, hw), f32),
        grid=(nblk,),
        in_specs=[
            smem,
            smem,
            img_spec,
            img_spec,
            full((4, 1, hw)),
            full((3 * P * hid, 3 * gr)),
            full((P * hid, 1)),
            full((P * hid, 1)),
            full((3 * gr, 3 * P * hid)),
            full((gr, 1)),
            full((gr, 1)),
        ],
        out_specs=img_spec,
        scratch_shapes=[
            pltpu.VMEM((3 * gr, hw), jnp.float32),
            pltpu.VMEM((3 * gr, hw), jnp.float32),
            pltpu.VMEM((3 * P * hid, hw), jnp.float32),
            pltpu.VMEM((3 * P * hid, hw), jnp.float32),
            pltpu.VMEM((P * hid, hw), jnp.float32),
            pltpu.VMEM((P * hid, hw), jnp.float32),
            pltpu.VMEM((gr, hw), jnp.float32),
            pltpu.VMEM((gr, hw), jnp.float32),
        ],
        compiler_params=pltpu.CompilerParams(
            dimension_semantics=("parallel",)),
    )(alpha_fd, alphas, clean2, noisy2, masks,
      wg1, b1p, temb1p, wg2, b2p, temb2p)

    return out.reshape(n, c, h, w)


# speedup vs baseline: 3.0765x; 1.0407x over previous
import functools

import numpy as np
import jax
import jax.numpy as jnp
from jax.experimental import pallas as pl
from jax.experimental.pallas import tpu as pltpu


def _diff_kernel(alpha_fd_ref, alphas_ref, clean_ref, noisy_ref, mask_ref,
                 wg1_ref, b1_ref, temb1_ref, wg2_ref, b2_ref, temb2_ref,
                 o_ref, s1_ref, s2_ref, hb_ref, x_ref,
                 *, width, timesteps, ngroups):
    hw = clean_ref.shape[2]
    gr = clean_ref.shape[1] // ngroups
    hid2 = wg1_ref.shape[0] // 3

    a_fd = alpha_fd_ref[0]

    wg1 = wg1_ref[...]
    wg2 = wg2_ref[...]
    b1 = b1_ref[...]
    temb1 = temb1_ref[...]
    b2 = b2_ref[...]
    temb2 = temb2_ref[...]
    mask_x0 = mask_ref[0]
    mask_x1 = mask_ref[1]

    groups = []
    for gi in range(ngroups):
        ro = gi * gr
        anc = (a_fd * noisy_ref[0, ro:ro + gr, :]
               + (1.0 - a_fd) * clean_ref[0, ro:ro + gr, :])
        groups.append((gi, ro, anc))

    def step(i, _):
        a_t = alphas_ref[i, 0]
        a_p = alphas_ref[i, 1]
        st = 1.0 - a_t
        sp = 1.0 - a_p
        v_p = b1 + a_p * temb1
        v_t = b1 + a_t * temb1
        cb = (a_t - a_p) * b2 + (a_t * a_t - a_t + a_p - a_p * a_p) * temb2
        for gi, ro, anc in groups:
            so1 = gi * 3 * gr
            so2 = gi * 3 * hid2
            ho = gi * hid2
            x = x_ref[ro:ro + gr, :]
            s1_ref[so1:so1 + gr, :] = pltpu.roll(x, shift=1, axis=1) * mask_x0
            s1_ref[so1 + gr:so1 + 2 * gr, :] = x
            s1_ref[so1 + 2 * gr:so1 + 3 * gr, :] = (
                pltpu.roll(x, shift=hw - 1, axis=1) * mask_x1)
            g1 = jnp.dot(wg1, s1_ref[so1:so1 + 3 * gr, :],
                         preferred_element_type=jnp.float32)
            hb_ref[ho:ho + hid2, :] = (
                g1[hid2:2 * hid2, :]
                + pltpu.roll(g1[0:hid2, :], shift=width, axis=1)
                + pltpu.roll(g1[2 * hid2:3 * hid2, :],
                             shift=hw - width, axis=1))
            hb_ref[ho:ho + hid2, 0:width] = (
                g1[hid2:2 * hid2, 0:width]
                + g1[2 * hid2:3 * hid2, width:2 * width])
            hb_ref[ho:ho + hid2, hw - width:hw] = (
                g1[hid2:2 * hid2, hw - width:hw]
                + g1[0:hid2, hw - 2 * width:hw - width])
            hb = hb_ref[ho:ho + hid2, :]
            d = (sp * jnp.maximum(hb + v_p, 0.0)
                 - st * jnp.maximum(hb + v_t, 0.0))
            s2_ref[so2:so2 + hid2, :] = pltpu.roll(d, shift=1, axis=1) * mask_x0
            s2_ref[so2 + hid2:so2 + 2 * hid2, :] = d
            s2_ref[so2 + 2 * hid2:so2 + 3 * hid2, :] = (
                pltpu.roll(d, shift=hw - 1, axis=1) * mask_x1)
            g = jnp.dot(wg2, s2_ref[so2:so2 + 3 * hid2, :],
                        preferred_element_type=jnp.float32)
            base = x + ((g[gr:2 * gr, :] + cb) + (a_p - a_t) * anc)
            x_ref[ro:ro + gr, :] = (
                base
                + pltpu.roll(g[0:gr, :], shift=width, axis=1)
                + pltpu.roll(g[2 * gr:3 * gr, :], shift=hw - width, axis=1))
            x_ref[ro:ro + gr, 0:width] = (
                base[:, 0:width] + g[2 * gr:3 * gr, width:2 * width])
            x_ref[ro:ro + gr, hw - width:hw] = (
                base[:, hw - width:hw]
                + g[0:gr, hw - 2 * width:hw - width])
        return 0

    for gi, ro, anc in groups:
        x_ref[ro:ro + gr, :] = anc
    jax.lax.fori_loop(0, timesteps, step, 0)
    o_ref[...] = x_ref[...][None].astype(o_ref.dtype)


def _make_edge_masks(h, w):
    ys, xs = np.divmod(np.arange(h * w), w)
    masks = np.stack([xs - 1 >= 0, xs + 1 < w, ys - 1 >= 0, ys + 1 < h]
                     ).astype(np.float32)[:, None, :]
    return jnp.asarray(masks)


def kernel(clean, noisy, t, w1_2d, b1, temb1, w2_big, b2, temb2, alphas):
    n, c, h, w = clean.shape
    hw = h * w
    hid = b1.shape[0]
    timesteps = alphas.shape[0]
    P = 4
    G = 4
    gr = P * c
    rows = G * gr
    nblk = n // (P * G)

    f32 = jnp.float32
    clean2 = clean.reshape(nblk, rows, hw).astype(f32)
    noisy2 = noisy.reshape(nblk, rows, hw).astype(f32)
    alpha_fd = jnp.reshape(jnp.asarray(t, f32) / timesteps, (1,))
    masks = _make_edge_masks(h, w)

    w1t = jnp.transpose(w1_2d.reshape(hid, 3, 3, 2 * c)[:, :, :, :c],
                        (1, 0, 2, 3))
    wg1_6d = jnp.zeros((3, P, hid, 3, P, c), f32)
    for j in range(P):
        wg1_6d = wg1_6d.at[:, j, :, :, j, :].set(w1t)
    wg1 = wg1_6d.reshape(3 * P * hid, 3 * P * c)

    w2t = jnp.transpose(w2_big[:c].reshape(c, 3, 3, 2 * hid)[:, :, :, :hid],
                        (1, 0, 2, 3))
    wg2_6d = jnp.zeros((3, P, c, 3, P, hid), f32)
    for j in range(P):
        wg2_6d = wg2_6d.at[:, j, :, :, j, :].set(w2t)
    wg2 = wg2_6d.reshape(3 * P * c, 3 * P * hid)

    b1p = jnp.tile(b1, (P, 1))
    temb1p = jnp.tile(temb1, (P, 1))
    b2p = jnp.tile(b2[:c], (P, 1))
    temb2p = jnp.tile(temb2[:c], (P, 1))

    img_spec = pl.BlockSpec((1, rows, hw), lambda i: (i, 0, 0))
    smem = pl.BlockSpec(memory_space=pltpu.MemorySpace.SMEM)

    def full(shape):
        return pl.BlockSpec(shape, lambda i: (0,) * len(shape))

    kern = functools.partial(_diff_kernel, width=w, timesteps=timesteps,
                             ngroups=G)
    out = pl.pallas_call(
        kern,
        out_shape=jax.ShapeDtypeStruct((nblk, rows, hw), f32),
        grid=(nblk,),
        in_specs=[
            smem,
            smem,
            img_spec,
            img_spec,
            full((4, 1, hw)),
            full((3 * P * hid, 3 * gr)),
            full((P * hid, 1)),
            full((P * hid, 1)),
            full((3 * gr, 3 * P * hid)),
            full((gr, 1)),
            full((gr, 1)),
        ],
        out_specs=img_spec,
        scratch_shapes=[
            pltpu.VMEM((G * 3 * gr, hw), jnp.float32),
            pltpu.VMEM((G * 3 * P * hid, hw), jnp.float32),
            pltpu.VMEM((G * P * hid, hw), jnp.float32),
            pltpu.VMEM((G * gr, hw), jnp.float32),
        ],
        compiler_params=pltpu.CompilerParams(
            dimension_semantics=("parallel",)),
    )(alpha_fd, alphas, clean2, noisy2, masks,
      wg1, b1p, temb1p, wg2, b2p, temb2p)

    return out.reshape(n, c, h, w)
